# baseline (speedup 1.0000x reference)
import jax
import jax.numpy as jnp
from jax.experimental import pallas as pl
from jax.experimental.pallas import tpu as pltpu


def _copy_body(x_ref, o_ref, sem_a, sem_b):
    ca = pltpu.make_async_copy(x_ref.at[pl.ds(0, 1)], o_ref.at[pl.ds(0, 1)], sem_a)
    cb = pltpu.make_async_copy(x_ref.at[pl.ds(2, 2)], o_ref.at[pl.ds(1, 2)], sem_b)
    ca.start()
    cb.start()
    ca.wait()
    cb.wait()


def kernel(x):
    return pl.pallas_call(
        _copy_body,
        out_shape=jax.ShapeDtypeStruct((3,) + x.shape[1:], x.dtype),
        in_specs=[pl.BlockSpec(memory_space=pl.ANY)],
        out_specs=pl.BlockSpec(memory_space=pl.ANY),
        scratch_shapes=[pltpu.SemaphoreType.DMA, pltpu.SemaphoreType.DMA],
    )(x)


# speedup vs baseline: 47.6946x; 47.6946x over previous
import jax
import jax.numpy as jnp
from jax.experimental import pallas as pl
from jax.experimental.pallas import tpu as pltpu

_ROWS = 4096
_COLS = 2048
_TILE = 512


def _copy_body(x_ref, o_ref):
    o_ref[...] = x_ref[...]


def kernel(x):
    def in_map(i, j):
        src = jnp.where(i >= 1, i + 1, i)
        return (src, j, 0)

    return pl.pallas_call(
        _copy_body,
        out_shape=jax.ShapeDtypeStruct((3, _ROWS, _COLS), x.dtype),
        grid=(3, _ROWS // _TILE),
        in_specs=[pl.BlockSpec((1, _TILE, _COLS), in_map)],
        out_specs=pl.BlockSpec((1, _TILE, _COLS), lambda i, j: (i, j, 0)),
    )(x)
